# baseline (speedup 1.0000x reference)
import functools

import jax
import jax.numpy as jnp
from jax.experimental import pallas as pl


def _mm_kernel(x_ref, w_ref, o_ref):
    o_ref[...] = jnp.dot(x_ref[...], w_ref[...],
                         preferred_element_type=jnp.float32)


def _mm(x, w):
    n, d = x.shape
    h = w.shape[1]
    blk = 1000
    return pl.pallas_call(
        _mm_kernel,
        grid=(n // blk,),
        in_specs=[pl.BlockSpec((blk, d), lambda i: (i, 0)),
                  pl.BlockSpec((d, h), lambda i: (0, 0))],
        out_specs=pl.BlockSpec((blk, h), lambda i: (i, 0)),
        out_shape=jax.ShapeDtypeStruct((n, h), jnp.float32),
    )(x, w)


def _gat_conv(x_src, x_dst, edge_index, p, self_loops):
    h_src = _mm(x_src, p["W_src"])
    h_dst = _mm(x_dst, p["W_dst"])
    src = edge_index[0]
    dst = edge_index[1]
    n_dst = x_dst.shape[0]
    if self_loops:
        idx = jnp.arange(n_dst, dtype=edge_index.dtype)
        src = jnp.concatenate([src, idx])
        dst = jnp.concatenate([dst, idx])
    a_src = h_src @ p["att_src"]
    a_dst = h_dst @ p["att_dst"]
    e = jax.nn.leaky_relu(a_src[src] + a_dst[dst], 0.2)
    m = jax.lax.stop_gradient(jax.ops.segment_max(e, dst, num_segments=n_dst))
    m = jnp.where(jnp.isfinite(m), m, 0.0)
    ex = jnp.exp(e - m[dst])
    den = jax.ops.segment_sum(ex, dst, num_segments=n_dst)
    alpha = ex / (den[dst] + 1e-16)
    out = jax.ops.segment_sum(h_src[src] * alpha[:, None], dst,
                              num_segments=n_dst)
    return out + p["bias"]


def _hetero_layer(xd, xt, params, ei_dd, ei_dt, ei_rev_dt, ei_tt):
    d = (_gat_conv(xd, xd, ei_dd, params["dd"], True)
         + _gat_conv(xt, xd, ei_rev_dt, params["rev"], False))
    t = (_gat_conv(xd, xt, ei_dt, params["dt"], False)
         + _gat_conv(xt, xt, ei_tt, params["tt"], True))
    return jax.nn.relu(d), jax.nn.relu(t)


def kernel(x_drug, x_target, params, ei_dd, ei_dt, ei_rev_dt, ei_tt,
           mask_d, mask_t):
    d1, t1 = _hetero_layer(x_drug, x_target, params, ei_dd, ei_dt,
                           ei_rev_dt, ei_tt)
    md = mask_d.astype(jnp.float32)
    mt = mask_t.astype(jnp.float32)
    xd_masked = x_drug * (1.0 - md) + md * params["mask_drug"]
    xt_masked = x_target * (1.0 - mt) + mt * params["mask_target"]
    d2, t2 = _hetero_layer(xd_masked, xt_masked, params, ei_dd, ei_dt,
                           ei_rev_dt, ei_tt)
    loss_d = jnp.sum(((d1 - d2) ** 2) * md) / (jnp.sum(md) * d1.shape[1] + 1e-8)
    loss_t = jnp.sum(((t1 - t2) ** 2) * mt) / (jnp.sum(mt) * t1.shape[1] + 1e-8)
    return d1, t1, loss_d + loss_t
